# baseline (speedup 1.0000x reference)
import functools

import jax
import jax.numpy as jnp
from jax import lax
from jax.experimental import pallas as pl
from jax.experimental.pallas import tpu as pltpu
from jax.experimental.pallas import tpu_sc as plsc

N = 10000
D = 128
E = 320000
NPAD = 10240
NCORES = 2
NTILES = 16
NW = NCORES * NTILES
CHUNK = 80
CPW = 125
CPM = 124
CPWP = 128
NBUF = 4
RPT = N // NTILES
DRPT = NPAD // NTILES

_mesh = plsc.VectorSubcoreMesh(core_axis_name="c", subcore_axis_name="s",
                               num_cores=NCORES, num_subcores=NTILES)


@functools.partial(
    pl.kernel,
    out_type=jax.ShapeDtypeStruct((NCORES, NPAD), jnp.float32),
    mesh=_mesh,
    scratch_types=[
        pltpu.VMEM((CPWP, CHUNK), jnp.int32),
        pltpu.VMEM((CHUNK,), jnp.float32),
        pltpu.VMEM((DRPT,), jnp.float32),
        pltpu.VMEM_SHARED((NPAD,), jnp.float32),
        pltpu.SemaphoreType.DMA,
        pltpu.SemaphoreType.DMA,
    ],
)
def _sc_degree(dst_hbm, out_hbm, dstv, onesv, zv, acc, semA, semB):
    c = lax.axis_index("c")
    s = lax.axis_index("s")
    w = c * NTILES + s
    pltpu.sync_copy(dst_hbm.at[w], dstv)

    def fill_ones(i, carry):
        onesv[pl.ds(i * 16, 16)] = jnp.full((16,), 1.0, jnp.float32)
        return carry

    lax.fori_loop(0, CHUNK // 16, fill_ones, 0)

    def fill_zero(i, carry):
        zv[pl.ds(i * 16, 16)] = jnp.zeros((16,), jnp.float32)
        return carry

    lax.fori_loop(0, DRPT // 16, fill_zero, 0)
    pltpu.sync_copy(zv, acc.at[pl.ds(s * DRPT, DRPT)])
    plsc.subcore_barrier()

    GRP = 5
    NGRP = CPW // GRP
    sems = (semA, semB)

    def issue_grp(gid, sem):
        def ib(t, carry):
            pltpu.async_copy(onesv, acc.at[dstv.at[gid * GRP + t]], sem,
                             add=True)
            return carry
        lax.fori_loop(0, GRP, ib, 0)

    def drain_grp(sem):
        def db(t, carry):
            pltpu.make_async_copy(onesv, acc.at[dstv.at[0]], sem).wait()
            return carry
        lax.fori_loop(0, GRP, db, 0)

    issue_grp(0, sems[0])
    issue_grp(1, sems[1])
    for gid in range(2, NGRP):
        drain_grp(sems[gid % 2])
        issue_grp(gid, sems[gid % 2])
    drain_grp(sems[NGRP % 2])
    drain_grp(sems[(NGRP + 1) % 2])
    plsc.subcore_barrier()
    pltpu.sync_copy(acc.at[pl.ds(s * DRPT, DRPT)],
                    out_hbm.at[c, pl.ds(s * DRPT, DRPT)])


@functools.partial(
    pl.kernel,
    out_type=jax.ShapeDtypeStruct((NCORES, N, D), jnp.float32),
    mesh=_mesh,
    scratch_types=[
        pltpu.VMEM((NBUF, CHUNK), jnp.int32),
        pltpu.VMEM((NBUF, CHUNK), jnp.int32),
        pltpu.VMEM((CHUNK, D), jnp.float32),
        pltpu.VMEM((CHUNK, D), jnp.float32),
        pltpu.VMEM((CHUNK, D), jnp.float32),
        pltpu.VMEM((CHUNK, D), jnp.float32),
        pltpu.VMEM_SHARED((N, D), jnp.float32),
        [pltpu.SemaphoreType.DMA] * NBUF,
        [pltpu.SemaphoreType.DMA] * NBUF,
        [pltpu.SemaphoreType.DMA] * NBUF,
        [pltpu.SemaphoreType.DMA] * NBUF,
    ],
)
def _sc_scatter(hn_hbm, src_hbm, dst_hbm, out_hbm,
                ring, dring, buf0, buf1, buf2, buf3, acc,
                gsems, ssems, isems, dsems):
    bufs = (buf0, buf1, buf2, buf3)
    c = lax.axis_index("c")
    s = lax.axis_index("s")
    w = c * NTILES + s

    for t in range(NBUF):
        pltpu.async_copy(src_hbm.at[w, t], ring.at[t], isems[t])
        pltpu.async_copy(dst_hbm.at[w, t], dring.at[t], dsems[t])
    for t in range(2):
        pltpu.make_async_copy(src_hbm.at[w, 0], ring.at[t],
                              isems[t]).wait()
        pltpu.async_copy(hn_hbm.at[ring.at[t]], bufs[t], gsems[t])

    def fill_zero(i, carry):
        buf3[i // 8, pl.ds((i % 8) * 16, 16)] = jnp.zeros((16,), jnp.float32)
        return carry

    lax.fori_loop(0, CHUNK * (D // 16), fill_zero, 0)

    @pl.when(s < NTILES - 1)
    def _():
        for k in range(640 // CHUNK):
            pltpu.sync_copy(buf3, acc.at[pl.ds(s * 640 + k * CHUNK, CHUNK)])

    @pl.when(s == NTILES - 1)
    def _():
        for k in range(400 // CHUNK):
            pltpu.sync_copy(buf3, acc.at[pl.ds(9600 + k * CHUNK, CHUNK)])

    plsc.subcore_barrier()

    def body(g, carry):
        for b in range(NBUF):
            j = g * NBUF + b
            bp = (b + 2) % NBUF

            @pl.when((j >= NBUF - 2) & (j + 2 < CPM))
            def _():
                pltpu.make_async_copy(bufs[bp], acc.at[dring.at[0]],
                                      ssems[bp]).wait()

            @pl.when((j >= NBUF - 2) & (j + 2 < CPM))
            def _():
                pltpu.async_copy(dst_hbm.at[w, j + 2], dring.at[bp],
                                 dsems[bp])

            @pl.when(j + 2 < CPM)
            def _():
                pltpu.make_async_copy(src_hbm.at[w, 0], ring.at[bp],
                                      isems[bp]).wait()
                pltpu.async_copy(hn_hbm.at[ring.at[bp]], bufs[bp], gsems[bp])

            pltpu.make_async_copy(hn_hbm.at[ring.at[b]], bufs[b],
                                  gsems[b]).wait()

            @pl.when(j + NBUF < CPM)
            def _():
                pltpu.async_copy(src_hbm.at[w, j + NBUF], ring.at[b],
                                 isems[b])

            pltpu.make_async_copy(dst_hbm.at[w, 0], dring.at[b],
                                  dsems[b]).wait()
            pltpu.async_copy(bufs[b], acc.at[dring.at[b]], ssems[b], add=True)
        return carry

    lax.fori_loop(0, CPM // NBUF, body, 0)
    for b in range(NBUF):
        pltpu.make_async_copy(bufs[b], acc.at[dring.at[0]], ssems[b]).wait()

    pltpu.sync_copy(src_hbm.at[w, CPM], ring.at[0])
    pltpu.sync_copy(dst_hbm.at[w, CPM], dring.at[0])
    pltpu.async_copy(hn_hbm.at[ring.at[0]], buf0, gsems[0])
    pltpu.make_async_copy(hn_hbm.at[ring.at[0]], buf0, gsems[0]).wait()
    pltpu.sync_copy(buf0, acc.at[dring.at[0]], add=True)

    plsc.subcore_barrier()

    @pl.when(s < NTILES - 1)
    def _():
        pltpu.sync_copy(acc.at[pl.ds(s * 640, 640)],
                        out_hbm.at[c, pl.ds(s * 640, 640)])

    @pl.when(s == NTILES - 1)
    def _():
        pltpu.sync_copy(acc.at[pl.ds(9600, 400)],
                        out_hbm.at[c, pl.ds(9600, 400)])


def _tc_mm_body(x_ref, w_ref, out_ref):
    out_ref[...] = jnp.dot(x_ref[...], w_ref[...],
                           preferred_element_type=jnp.float32)


_tc_mm = pl.pallas_call(
    _tc_mm_body,
    out_shape=jax.ShapeDtypeStruct((N, D), jnp.float32),
)


def _tc_scale_body(h_ref, d0_ref, d1_ref, hn_ref, dinv_ref):
    deg = d0_ref[...] + d1_ref[...] + 1.0
    dinv = lax.rsqrt(deg)[:N]
    hn_ref[...] = h_ref[...] * dinv
    dinv_ref[...] = dinv


_tc_scale = pl.pallas_call(
    _tc_scale_body,
    out_shape=(jax.ShapeDtypeStruct((N, D), jnp.float32),
               jax.ShapeDtypeStruct((N, 1), jnp.float32)),
)


GB = 10
BR = N // GB


def _tc_mid_body(p_ref, hn_ref, dinv_ref, b_ref, g_ref, bt_ref, w_ref,
                 out_ref, a_scr, st_scr):
    i = pl.program_id(0)

    @pl.when(i == 0)
    def _():
        st_scr[...] = jnp.zeros_like(st_scr)

    @pl.when(i < GB)
    def _():
        a = dinv_ref[...] * (p_ref[0] + p_ref[1] + hn_ref[...]) + b_ref[...]
        a_scr[pl.ds(i * BR, BR), :] = a
        st_scr[0:1, :] += jnp.sum(a, axis=0, keepdims=True)
        st_scr[1:2, :] += jnp.sum(a * a, axis=0, keepdims=True)
        out_ref[...] = a

    @pl.when(i >= GB)
    def _():
        ib = i - GB
        a = a_scr[pl.ds(ib * BR, BR), :]
        mu = st_scr[0:1, :] * (1.0 / N)
        var = st_scr[1:2, :] * (1.0 / N) - mu * mu
        h = jnp.maximum(
            (a - mu) * lax.rsqrt(var + 1e-5) * g_ref[...] + bt_ref[...], 0.0)
        out_ref[...] = jnp.dot(h, w_ref[...],
                               preferred_element_type=jnp.float32) * dinv_ref[...]


def _ph0_idx(i):
    return (0, jnp.where(i < GB, i, GB - 1), 0)


def _row_idx(i):
    return (jnp.where(i < GB, i, i - GB), 0)


_tc_mid = pl.pallas_call(
    _tc_mid_body,
    grid=(2 * GB,),
    in_specs=[pl.BlockSpec((NCORES, BR, D), _ph0_idx),
              pl.BlockSpec((BR, D), lambda i: (jnp.where(i < GB, i, GB - 1), 0)),
              pl.BlockSpec((BR, 1), _row_idx),
              pl.BlockSpec((1, D), lambda i: (0, 0)),
              pl.BlockSpec((1, D), lambda i: (0, 0)),
              pl.BlockSpec((1, D), lambda i: (0, 0)),
              pl.BlockSpec((D, D), lambda i: (0, 0))],
    out_specs=pl.BlockSpec((BR, D), _row_idx),
    out_shape=jax.ShapeDtypeStruct((N, D), jnp.float32),
    scratch_shapes=[pltpu.VMEM((N, D), jnp.float32),
                    pltpu.VMEM((2, D), jnp.float32)],
)


def _tc_final_body(p_ref, hn_ref, dinv_ref, b_ref, out_ref):
    out_ref[...] = (dinv_ref[...] * (p_ref[0] + p_ref[1] + hn_ref[...])
                    + b_ref[...])


_tc_final = pl.pallas_call(
    _tc_final_body,
    out_shape=jax.ShapeDtypeStruct((N, D), jnp.float32),
)


def kernel(x, edge_index, W1, b1, g1, bt1, W2, b2, g2, bt2, W3, b3):
    srcp = jnp.pad(edge_index[0].reshape(NW, CPW, CHUNK),
                   ((0, 0), (0, CPWP - CPW), (0, 0)))
    dstp = jnp.pad(edge_index[1].reshape(NW, CPW, CHUNK),
                   ((0, 0), (0, CPWP - CPW), (0, 0)))

    h1 = _tc_mm(x, W1)
    degp = _sc_degree(dstp)
    d0 = degp[0].reshape(NPAD, 1)
    d1 = degp[1].reshape(NPAD, 1)

    hn1, dinv = _tc_scale(h1, d0, d1)
    p1 = _sc_scatter(hn1, srcp, dstp)
    hn2 = _tc_mid(p1, hn1, dinv, b1.reshape(1, D), g1.reshape(1, D),
                  bt1.reshape(1, D), W2)
    p2 = _sc_scatter(hn2, srcp, dstp)
    hn3 = _tc_mid(p2, hn2, dinv, b2.reshape(1, D), g2.reshape(1, D),
                  bt2.reshape(1, D), W3)
    p3 = _sc_scatter(hn3, srcp, dstp)
    return _tc_final(p3, hn3, dinv, b3.reshape(1, D))


# speedup vs baseline: 1.0360x; 1.0360x over previous
import functools

import jax
import jax.numpy as jnp
from jax import lax
from jax.experimental import pallas as pl
from jax.experimental.pallas import tpu as pltpu
from jax.experimental.pallas import tpu_sc as plsc

N = 10000
D = 128
E = 320000
NPAD = 10240
NCORES = 2
NTILES = 16
NW = NCORES * NTILES
CHUNK = 80
CPW = 125
CPM = 124
CPWP = 128
NBUF = 4
RPT = N // NTILES
DRPT = NPAD // NTILES

_mesh = plsc.VectorSubcoreMesh(core_axis_name="c", subcore_axis_name="s",
                               num_cores=NCORES, num_subcores=NTILES)


@functools.partial(
    pl.kernel,
    out_type=jax.ShapeDtypeStruct((NCORES, NPAD), jnp.float32),
    mesh=_mesh,
    scratch_types=[
        pltpu.VMEM((CPWP, CHUNK), jnp.int32),
        pltpu.VMEM((CHUNK,), jnp.float32),
        pltpu.VMEM((DRPT,), jnp.float32),
        pltpu.VMEM_SHARED((NPAD,), jnp.float32),
        pltpu.SemaphoreType.DMA,
        pltpu.SemaphoreType.DMA,
    ],
)
def _sc_degree(dst_hbm, out_hbm, dstv, onesv, zv, acc, semA, semB):
    c = lax.axis_index("c")
    s = lax.axis_index("s")
    w = c * NTILES + s
    pltpu.sync_copy(dst_hbm.at[w], dstv)

    def fill_ones(i, carry):
        onesv[pl.ds(i * 16, 16)] = jnp.full((16,), 1.0, jnp.float32)
        return carry

    lax.fori_loop(0, CHUNK // 16, fill_ones, 0)

    def fill_zero(i, carry):
        zv[pl.ds(i * 16, 16)] = jnp.zeros((16,), jnp.float32)
        return carry

    lax.fori_loop(0, DRPT // 16, fill_zero, 0)
    pltpu.sync_copy(zv, acc.at[pl.ds(s * DRPT, DRPT)])
    plsc.subcore_barrier()

    GRP = 5
    NGRP = CPW // GRP
    sems = (semA, semB)

    def issue_grp(gid, sem):
        def ib(t, carry):
            pltpu.async_copy(onesv, acc.at[dstv.at[gid * GRP + t]], sem,
                             add=True)
            return carry
        lax.fori_loop(0, GRP, ib, 0)

    def drain_grp(sem):
        def db(t, carry):
            pltpu.make_async_copy(onesv, acc.at[dstv.at[0]], sem).wait()
            return carry
        lax.fori_loop(0, GRP, db, 0)

    issue_grp(0, sems[0])
    issue_grp(1, sems[1])
    for gid in range(2, NGRP):
        drain_grp(sems[gid % 2])
        issue_grp(gid, sems[gid % 2])
    drain_grp(sems[NGRP % 2])
    drain_grp(sems[(NGRP + 1) % 2])
    plsc.subcore_barrier()
    pltpu.sync_copy(acc.at[pl.ds(s * DRPT, DRPT)],
                    out_hbm.at[c, pl.ds(s * DRPT, DRPT)])


@functools.partial(
    pl.kernel,
    out_type=jax.ShapeDtypeStruct((NCORES, N, D), jnp.float32),
    mesh=_mesh,
    scratch_types=[
        pltpu.VMEM((NBUF, CHUNK), jnp.int32),
        pltpu.VMEM((NBUF, CHUNK), jnp.int32),
        pltpu.VMEM((CHUNK, D), jnp.float32),
        pltpu.VMEM((CHUNK, D), jnp.float32),
        pltpu.VMEM((CHUNK, D), jnp.float32),
        pltpu.VMEM((CHUNK, D), jnp.float32),
        pltpu.VMEM_SHARED((N, D), jnp.float32),
        [pltpu.SemaphoreType.DMA] * NBUF,
        [pltpu.SemaphoreType.DMA] * NBUF,
        [pltpu.SemaphoreType.DMA] * NBUF,
        [pltpu.SemaphoreType.DMA] * NBUF,
    ],
)
def _sc_scatter(hn_hbm, src_hbm, dst_hbm, out_hbm,
                ring, dring, buf0, buf1, buf2, buf3, acc,
                gsems, ssems, isems, dsems):
    bufs = (buf0, buf1, buf2, buf3)
    c = lax.axis_index("c")
    s = lax.axis_index("s")
    w = c * NTILES + s

    for t in range(NBUF):
        pltpu.async_copy(src_hbm.at[w, t], ring.at[t], isems[t])
        pltpu.async_copy(dst_hbm.at[w, t], dring.at[t], dsems[t])
    for t in range(2):
        pltpu.make_async_copy(src_hbm.at[w, 0], ring.at[t],
                              isems[t]).wait()
        pltpu.async_copy(hn_hbm.at[ring.at[t]], bufs[t], gsems[t])

    def fill_zero(i, carry):
        buf3[i // 8, pl.ds((i % 8) * 16, 16)] = jnp.zeros((16,), jnp.float32)
        return carry

    lax.fori_loop(0, CHUNK * (D // 16), fill_zero, 0)

    @pl.when(s < NTILES - 1)
    def _():
        for k in range(640 // CHUNK):
            pltpu.sync_copy(buf3, acc.at[pl.ds(s * 640 + k * CHUNK, CHUNK)])

    @pl.when(s == NTILES - 1)
    def _():
        for k in range(400 // CHUNK):
            pltpu.sync_copy(buf3, acc.at[pl.ds(9600 + k * CHUNK, CHUNK)])

    plsc.subcore_barrier()

    def body(g, carry):
        for b in range(NBUF):
            j = g * NBUF + b
            bp = (b + 2) % NBUF

            @pl.when((j >= NBUF - 2) & (j + 2 < CPM))
            def _():
                pltpu.make_async_copy(bufs[bp], acc.at[dring.at[0]],
                                      ssems[bp]).wait()

            @pl.when((j >= NBUF - 2) & (j + 2 < CPM))
            def _():
                pltpu.async_copy(dst_hbm.at[w, j + 2], dring.at[bp],
                                 dsems[bp])

            @pl.when(j + 2 < CPM)
            def _():
                pltpu.make_async_copy(src_hbm.at[w, 0], ring.at[bp],
                                      isems[bp]).wait()
                pltpu.async_copy(hn_hbm.at[ring.at[bp]], bufs[bp], gsems[bp])

            pltpu.make_async_copy(hn_hbm.at[ring.at[b]], bufs[b],
                                  gsems[b]).wait()

            @pl.when(j + NBUF < CPM)
            def _():
                pltpu.async_copy(src_hbm.at[w, j + NBUF], ring.at[b],
                                 isems[b])

            pltpu.make_async_copy(dst_hbm.at[w, 0], dring.at[b],
                                  dsems[b]).wait()
            pltpu.async_copy(bufs[b], acc.at[dring.at[b]], ssems[b], add=True)
        return carry

    lax.fori_loop(0, CPM // NBUF, body, 0)
    for b in range(NBUF):
        pltpu.make_async_copy(bufs[b], acc.at[dring.at[0]], ssems[b]).wait()

    pltpu.sync_copy(src_hbm.at[w, CPM], ring.at[0])
    pltpu.sync_copy(dst_hbm.at[w, CPM], dring.at[0])
    pltpu.async_copy(hn_hbm.at[ring.at[0]], buf0, gsems[0])
    pltpu.make_async_copy(hn_hbm.at[ring.at[0]], buf0, gsems[0]).wait()
    pltpu.sync_copy(buf0, acc.at[dring.at[0]], add=True)

    plsc.subcore_barrier()

    @pl.when(s < NTILES - 1)
    def _():
        pltpu.sync_copy(acc.at[pl.ds(s * 640, 640)],
                        out_hbm.at[c, pl.ds(s * 640, 640)])

    @pl.when(s == NTILES - 1)
    def _():
        pltpu.sync_copy(acc.at[pl.ds(9600, 400)],
                        out_hbm.at[c, pl.ds(9600, 400)])


def _tc_mm_body(x_ref, w_ref, out_ref):
    out_ref[...] = jnp.dot(x_ref[...], w_ref[...],
                           preferred_element_type=jnp.float32)


_tc_mm = pl.pallas_call(
    _tc_mm_body,
    out_shape=jax.ShapeDtypeStruct((N, D), jnp.float32),
)


def _tc_scale_body(h_ref, d0_ref, d1_ref, hn_ref, dinv_ref):
    deg = d0_ref[...] + d1_ref[...] + 1.0
    dinv = lax.rsqrt(deg)[:N]
    hn_ref[...] = h_ref[...] * dinv
    dinv_ref[...] = dinv


_tc_scale = pl.pallas_call(
    _tc_scale_body,
    out_shape=(jax.ShapeDtypeStruct((N, D), jnp.float32),
               jax.ShapeDtypeStruct((N, 1), jnp.float32)),
)


def _tc_mid_body(p_ref, hn_ref, dinv_ref, b_ref, g_ref, bt_ref, w_ref,
                 out_ref):
    dinv = dinv_ref[...]
    a = dinv * (p_ref[0] + p_ref[1] + hn_ref[...]) + b_ref[...]
    mu = jnp.sum(a, axis=0, keepdims=True) * (1.0 / N)
    dev = a - mu
    var = jnp.sum(dev * dev, axis=0, keepdims=True) * (1.0 / N)
    hb = dev * lax.rsqrt(var + 1e-5) * g_ref[...] + bt_ref[...]
    h = jnp.maximum(hb, 0.0)
    out_ref[...] = jnp.dot(h, w_ref[...],
                           preferred_element_type=jnp.float32) * dinv


_tc_mid = pl.pallas_call(
    _tc_mid_body,
    out_shape=jax.ShapeDtypeStruct((N, D), jnp.float32),
)


def _tc_final_body(p_ref, hn_ref, dinv_ref, b_ref, out_ref):
    out_ref[...] = (dinv_ref[...] * (p_ref[0] + p_ref[1] + hn_ref[...])
                    + b_ref[...])


_tc_final = pl.pallas_call(
    _tc_final_body,
    out_shape=jax.ShapeDtypeStruct((N, D), jnp.float32),
)


def kernel(x, edge_index, W1, b1, g1, bt1, W2, b2, g2, bt2, W3, b3):
    srcp = jnp.pad(edge_index[0].reshape(NW, CPW, CHUNK),
                   ((0, 0), (0, CPWP - CPW), (0, 0)))
    dstp = jnp.pad(edge_index[1].reshape(NW, CPW, CHUNK),
                   ((0, 0), (0, CPWP - CPW), (0, 0)))

    h1 = _tc_mm(x, W1)
    degp = _sc_degree(dstp)
    d0 = degp[0].reshape(NPAD, 1)
    d1 = degp[1].reshape(NPAD, 1)

    hn1, dinv = _tc_scale(h1, d0, d1)
    p1 = _sc_scatter(hn1, srcp, dstp)
    hn2 = _tc_mid(p1, hn1, dinv, b1.reshape(1, D), g1.reshape(1, D),
                  bt1.reshape(1, D), W2)
    p2 = _sc_scatter(hn2, srcp, dstp)
    hn3 = _tc_mid(p2, hn2, dinv, b2.reshape(1, D), g2.reshape(1, D),
                  bt2.reshape(1, D), W3)
    p3 = _sc_scatter(hn3, srcp, dstp)
    return _tc_final(p3, hn3, dinv, b3.reshape(1, D))
